# Initial kernel scaffold; baseline (speedup 1.0000x reference)
import jax
import jax.numpy as jnp
from jax.experimental import pallas as pl


def kernel(feat, edge_index, W, b):
    raise NotImplementedError("write your pallas kernel here")


# speedup vs baseline: 3.6018x; 3.6018x over previous
import functools

import jax
import jax.numpy as jnp
from jax import lax
from jax.experimental import pallas as pl
from jax.experimental.pallas import tpu as pltpu
from jax.experimental.pallas import tpu_sc as plsc

N = 10000
E = 320000
D = 128

NC = 2
NS = 16
NW = NC * NS

K = 128
CH = 80
IB = 16
GROUPS = CH // IB
EPAD = NW * CH * K
NPAD = 10240
ROWS_PER_TILE = NPAD // NS


def _mm_body(x_ref, w_ref, o_ref):
    o_ref[...] = jnp.dot(x_ref[...], w_ref[...], preferred_element_type=jnp.float32)


def _matmul(feat, W):
    return pl.pallas_call(
        _mm_body,
        grid=(10,),
        in_specs=[
            pl.BlockSpec((1000, D), lambda i: (i, 0)),
            pl.BlockSpec((D, D), lambda i: (0, 0)),
        ],
        out_specs=pl.BlockSpec((1000, D), lambda i: (i, 0)),
        out_shape=jax.ShapeDtypeStruct((N, D), jnp.float32),
    )(feat, W)


def _combine_body(p0_ref, p1_ref, b_ref, o_ref):
    o_ref[...] = p0_ref[...] + p1_ref[...] + b_ref[...]


def _combine(p0, p1, b2d):
    return pl.pallas_call(
        _combine_body,
        grid=(10,),
        in_specs=[
            pl.BlockSpec((1000, D), lambda i: (i, 0)),
            pl.BlockSpec((1000, D), lambda i: (i, 0)),
            pl.BlockSpec((1, D), lambda i: (0, 0)),
        ],
        out_specs=pl.BlockSpec((1000, D), lambda i: (i, 0)),
        out_shape=jax.ShapeDtypeStruct((N, D), jnp.float32),
    )(p0, p1, b2d)


@functools.lru_cache(maxsize=None)
def _make_sc_kernel():
    mesh = plsc.VectorSubcoreMesh(core_axis_name="c", subcore_axis_name="s")

    @functools.partial(
        pl.kernel,
        mesh=mesh,
        out_type=jax.ShapeDtypeStruct((NC, NPAD, D), jnp.float32),
        scratch_types=[
            pltpu.VMEM((IB, K), jnp.int32),
            pltpu.VMEM((IB, K), jnp.int32),
            pltpu.VMEM((K, D), jnp.float32),
            pltpu.VMEM((K, D), jnp.float32),
            pltpu.VMEM_SHARED((NPAD, D), jnp.float32),
            pltpu.SemaphoreType.DMA,
            pltpu.SemaphoreType.DMA,
        ],
    )
    def sc_scatter(h_hbm, src_hbm, dst_hbm, out_hbm,
                   sidx, didx, buf0, buf1, accum, sem0, sem1):
        c = lax.axis_index("c")
        s = lax.axis_index("s")
        wid = c * NS + s

        zeros16 = jnp.zeros((16,), jnp.float32)

        def _zero_row(r, _):
            for col in range(D // 16):
                buf0[r, pl.ds(col * 16, 16)] = zeros16
            return _

        lax.fori_loop(0, K, _zero_row, None)
        for k in range(ROWS_PER_TILE // K):
            pltpu.sync_copy(buf0, accum.at[pl.ds(s * ROWS_PER_TILE + k * K, K)])
        plsc.subcore_barrier()

        def _group(gi, _):
            pltpu.sync_copy(src_hbm.at[wid, pl.ds(gi * IB, IB)], sidx)
            pltpu.sync_copy(dst_hbm.at[wid, pl.ds(gi * IB, IB)], didx)

            pltpu.async_copy(h_hbm.at[sidx.at[0]], buf0, sem0)

            def _pair(i, _):
                j = 2 * i
                pltpu.make_async_copy(h_hbm.at[sidx.at[j]], buf0, sem0).wait()
                pltpu.async_copy(h_hbm.at[sidx.at[j + 1]], buf1, sem1)
                pltpu.sync_copy(buf0, accum.at[didx.at[j]], add=True)
                pltpu.make_async_copy(h_hbm.at[sidx.at[j + 1]], buf1, sem1).wait()

                @pl.when(i < IB // 2 - 1)
                def _():
                    pltpu.async_copy(h_hbm.at[sidx.at[j + 2]], buf0, sem0)

                pltpu.sync_copy(buf1, accum.at[didx.at[j + 1]], add=True)
                return _

            lax.fori_loop(0, IB // 2, _pair, None)
            return _

        lax.fori_loop(0, GROUPS, _group, None)

        plsc.subcore_barrier()
        pltpu.sync_copy(accum.at[pl.ds(s * ROWS_PER_TILE, ROWS_PER_TILE)],
                        out_hbm.at[c, pl.ds(s * ROWS_PER_TILE, ROWS_PER_TILE)])

    return sc_scatter


def kernel(feat, edge_index, W, b):
    dst = edge_index[0].astype(jnp.int32)
    src = edge_index[1].astype(jnp.int32)
    pad = EPAD - E
    src_p = jnp.concatenate([src, jnp.zeros((pad,), jnp.int32)])
    dst_p = jnp.concatenate([dst, jnp.full((pad,), NPAD - 1, jnp.int32)])
    src3 = src_p.reshape(NW, CH, K)
    dst3 = dst_p.reshape(NW, CH, K)

    h = _matmul(feat, W)
    parts = _make_sc_kernel()(h, src3, dst3)
    out = _combine(parts[0], parts[1], b.reshape(1, D))
    return out
